# baseline (speedup 1.0000x reference)
import functools

import jax
import jax.numpy as jnp
from jax.experimental import pallas as pl
from jax.experimental.pallas import tpu as pltpu

QB = 512
LPAD = 1024


def _dist_argmin_kernel(nvalid, xx_ref, cm_ref, out_ref):
    xb = xx_ref[...]
    nrm = jnp.sqrt(jnp.sum(xb * xb, axis=1, keepdims=True))
    fea = xb / nrm
    dot = jnp.dot(fea, cm_ref[...], preferred_element_type=jnp.float32)
    cm = cm_ref[...]
    cn = jnp.sum(cm * cm, axis=0, keepdims=True)
    scores = cn - 2.0 * dot
    lane = jax.lax.broadcasted_iota(jnp.int32, scores.shape, 1)
    scores = jnp.where(lane < nvalid, scores, jnp.inf)
    pred = jnp.argmin(scores, axis=1).astype(jnp.int32)
    out_ref[0, 0, :] = pred


def kernel(x, initc, labelset):
    q, d = x.shape
    l = labelset.shape[0]
    centers = jnp.take(initc, labelset, axis=0)
    cmat = jnp.zeros((d + 1, LPAD), dtype=jnp.float32)
    cmat = cmat.at[:, :l].set(centers.T)
    xx = jnp.concatenate([x, jnp.ones((q, 1), dtype=x.dtype)], axis=1)

    nq = q // QB
    pred = pl.pallas_call(
        functools.partial(_dist_argmin_kernel, l),
        grid=(nq,),
        in_specs=[
            pl.BlockSpec((QB, d + 1), lambda i: (i, 0)),
            pl.BlockSpec((d + 1, LPAD), lambda i: (0, 0)),
        ],
        out_specs=pl.BlockSpec((1, 1, QB), lambda i: (i, 0, 0)),
        out_shape=jax.ShapeDtypeStruct((nq, 1, QB), jnp.int32),
    )(xx, cmat)
    pred = pred.reshape(q)
    return jnp.take(labelset, pred, axis=0)


# speedup vs baseline: 1.1984x; 1.1984x over previous
import functools

import jax
import jax.numpy as jnp
from jax.experimental import pallas as pl
from jax.experimental.pallas import tpu as pltpu

QB = 512
LPAD = 1024


def _dist_argmin_kernel(nvalid, x_ref, cm_ref, out_ref, cn_ref):
    @pl.when(pl.program_id(0) == 0)
    def _():
        cm = cm_ref[...]
        cn_ref[...] = jnp.sum(cm * cm, axis=0, keepdims=True)

    xb = x_ref[...]
    feac = jnp.concatenate(
        [xb, jnp.ones((xb.shape[0], 1), dtype=xb.dtype)], axis=1)
    nrm = jnp.sqrt(jnp.sum(feac * feac, axis=1, keepdims=True))
    fea = feac / nrm
    dot = jnp.dot(fea, cm_ref[...], preferred_element_type=jnp.float32)
    scores = cn_ref[...] - 2.0 * dot
    lane = jax.lax.broadcasted_iota(jnp.int32, scores.shape, 1)
    scores = jnp.where(lane < nvalid, scores, jnp.inf)
    pred = jnp.argmin(scores, axis=1).astype(jnp.int32)
    out_ref[0, 0, :] = pred


def kernel(x, initc, labelset):
    q, d = x.shape
    l = labelset.shape[0]
    centers = jnp.take(initc, labelset, axis=0)
    cmat = jnp.zeros((d + 1, LPAD), dtype=jnp.float32)
    cmat = cmat.at[:, :l].set(centers.T)

    nq = q // QB
    pred = pl.pallas_call(
        functools.partial(_dist_argmin_kernel, l),
        grid=(nq,),
        in_specs=[
            pl.BlockSpec((QB, d), lambda i: (i, 0)),
            pl.BlockSpec((d + 1, LPAD), lambda i: (0, 0)),
        ],
        out_specs=pl.BlockSpec((1, 1, QB), lambda i: (i, 0, 0)),
        out_shape=jax.ShapeDtypeStruct((nq, 1, QB), jnp.int32),
        scratch_shapes=[pltpu.VMEM((1, LPAD), jnp.float32)],
    )(x, cmat)
    pred = pred.reshape(q)
    return jnp.take(labelset, pred, axis=0)
